# baseline (speedup 1.0000x reference)
import functools

import jax
import jax.numpy as jnp
from jax import lax
from jax.experimental import pallas as pl
from jax.experimental.pallas import tpu as pltpu
from jax.experimental.pallas import tpu_sc as plsc

BATCH = 16384
HIST = 200
EMBED_DIM = 32
NUM_CORES = 2
NUM_SUBCORES = 16
NW = NUM_CORES * NUM_SUBCORES
B_PER_W = BATCH // NW
NB = 4
SPLITS = ((0, 96), (96, 104))
NCHUNK = B_PER_W // NB
NBUF = 2


def _gather_kernel(idx_hbm, table_hbm, out_hbm,
                   idx_v, rows_v, gsem0, gsem1, osem0, osem1):
    wid = lax.axis_index("s") * NUM_CORES + lax.axis_index("c")
    batch0 = wid * B_PER_W
    gsems = (gsem0, gsem1)
    osems = (osem0, osem1)

    def fire(c, b):
        b0 = pl.multiple_of(batch0 + c * NB, NB)
        pltpu.sync_copy(idx_hbm.at[pl.ds(b0, NB)], idx_v.at[b])
        for i in range(NB):
            for h0, hn in SPLITS:
                pltpu.async_copy(
                    table_hbm.at[idx_v.at[b].at[i].at[pl.ds(h0, hn)]],
                    rows_v.at[b].at[i].at[pl.ds(h0, hn)],
                    gsems[b],
                )

    def drain_gathers(b):
        for i in range(NB):
            for h0, hn in SPLITS:
                pltpu.make_async_copy(
                    table_hbm.at[idx_v.at[b].at[i].at[pl.ds(h0, hn)]],
                    rows_v.at[b].at[i].at[pl.ds(h0, hn)],
                    gsems[b],
                ).wait()

    def writeout_start(c, b):
        b0 = pl.multiple_of(batch0 + c * NB, NB)
        pltpu.async_copy(rows_v.at[b], out_hbm.at[pl.ds(b0, NB)], osems[b])

    def writeout_wait(c, b):
        b0 = pl.multiple_of(batch0 + c * NB, NB)
        pltpu.make_async_copy(
            rows_v.at[b], out_hbm.at[pl.ds(b0, NB)], osems[b]
        ).wait()

    def pair_body(p, carry):
        c0 = p * NBUF
        for b in range(NBUF):

            @pl.when(p > 0)
            def _():
                writeout_wait((p - 1) * NBUF + b, b)

            fire(c0 + b, b)
        for b in range(NBUF):
            drain_gathers(b)
            writeout_start(c0 + b, b)
        return carry

    lax.fori_loop(0, NCHUNK // NBUF, pair_body, 0)
    for b in range(NBUF):
        writeout_wait(NCHUNK - NBUF + b, b)


@jax.jit
def _run(time_idx, table):
    mesh = plsc.VectorSubcoreMesh(core_axis_name="c", subcore_axis_name="s")
    kfn = functools.partial(
        pl.kernel,
        mesh=mesh,
        out_type=jax.ShapeDtypeStruct((BATCH, HIST, EMBED_DIM), jnp.float32),
        scratch_types=[
            pltpu.VMEM((NBUF, NB, HIST), jnp.int32),
            pltpu.VMEM((NBUF, NB, HIST, EMBED_DIM), jnp.float32),
            pltpu.SemaphoreType.DMA,
            pltpu.SemaphoreType.DMA,
            pltpu.SemaphoreType.DMA,
            pltpu.SemaphoreType.DMA,
        ],
        compiler_params=pltpu.CompilerParams(use_tc_tiling_on_sc=False),
    )(_gather_kernel)
    return kfn(time_idx, table)


def kernel(time_idx, table):
    return _run(time_idx, table)
